# baseline (speedup 1.0000x reference)
import numpy as np
import jax
import jax.numpy as jnp
from jax.experimental import pallas as pl
from jax.experimental.pallas import tpu as pltpu

_EPS = 1e-5
_DOT = jnp.bfloat16
_F32 = jnp.float32


def _cparams():
    return pltpu.CompilerParams(
        dimension_semantics=("parallel",),
        vmem_limit_bytes=48 * 1024 * 1024,
    )


def _norm_params(s_ref, ss_ref, g_ref, b_ref, cnt):
    mean = jnp.sum(s_ref[...], axis=(0, 1)) / cnt
    var = jnp.maximum(jnp.sum(ss_ref[...], axis=(0, 1)) / cnt - mean * mean, 0.0)
    scale = g_ref[...].reshape(-1) * jax.lax.rsqrt(var + _EPS)
    shift = b_ref[...].reshape(-1) - mean * scale
    return scale, shift


def _conv3x3(vals, buf_refs, w_refs):
    H, W, _ = vals[0].shape
    acc = None
    for v, buf_ref, w_ref in zip(vals, buf_refs, w_refs):
        C = v.shape[-1]
        vb = v.astype(buf_ref.dtype)
        buf_ref[...] = jnp.zeros_like(buf_ref)
        buf_ref[1:H + 1, 1:W, 0:C] = vb[:, 0:W - 1, :]
        buf_ref[1:H + 1, :, C:2 * C] = vb
        buf_ref[1:H + 1, 0:W - 1, 2 * C:3 * C] = vb[:, 1:W, :]
        for dy in range(3):
            p = buf_ref[dy:dy + H, :, :].reshape(H * W, 3 * C)
            d = jnp.dot(p, w_ref[dy * 3 * C:(dy + 1) * 3 * C, :],
                        preferred_element_type=_F32)
            acc = d if acc is None else acc + d
    return acc


def _write_out(acc, o_ref, so_ref, sso_ref, H, W):
    Cout = o_ref.shape[-1]
    o_ref[...] = acc.reshape(1, H, W, Cout).astype(o_ref.dtype)
    so_ref[...] = jnp.sum(acc, axis=0).reshape(1, 1, Cout)
    sso_ref[...] = jnp.sum(acc * acc, axis=0).reshape(1, 1, Cout)


def _k_first_conv(x_ref, w_ref, o_ref, so_ref, sso_ref, pad_ref):
    _, H, W, C = x_ref.shape
    acc = _conv3x3([x_ref[...].reshape(H, W, C)], [pad_ref], [w_ref])
    _write_out(acc, o_ref, so_ref, sso_ref, H, W)


def _make_norm_conv(pool_hw, cnt):
    def body(x_ref, s_ref, ss_ref, g_ref, b_ref, w_ref,
             o_ref, so_ref, sso_ref, pad_ref):
        _, H, W, C = x_ref.shape
        scale, shift = _norm_params(s_ref, ss_ref, g_ref, b_ref, cnt)
        y = jnp.maximum(x_ref[...].reshape(H, W, C).astype(_F32) * scale + shift,
                        0.0)
        if pool_hw is not None:
            Ho, Wo = pool_hw
            y = jnp.max(y.reshape(Ho, H // Ho, W, C), axis=1)
            y = jnp.max(y.reshape(Ho, Wo, W // Wo, C), axis=2)
        else:
            Ho, Wo = H, W
        acc = _conv3x3([y], [pad_ref], [w_ref])
        _write_out(acc, o_ref, so_ref, sso_ref, Ho, Wo)
    return body


def _make_dec_fused(bilinear, cnt_x, cnt_s):
    def body(*refs):
        (xr, xs, xss, xg, xb, wT, bT,
         skr, sks, skss, skg, skb) = refs[:12]
        i = 12
        if bilinear:
            a_ref = refs[i]
            i += 1
        w0, w1, o_ref, so_ref, sso_ref, pad0, pad1 = refs[i:i + 7]
        _, H, W, Cin = xr.shape
        CoutT = wT.shape[1] // 4
        scale, shift = _norm_params(xs, xss, xg, xb, cnt_x)
        xv = jnp.maximum(xr[...].reshape(H * W, Cin).astype(_F32) * scale + shift,
                         0.0)
        y = jnp.dot(xv.astype(_DOT), wT[...], preferred_element_type=_F32) + bT[...]
        y = y.reshape(H, W, 4, CoutT)
        r0 = y[:, :, 0:2, :].reshape(H, 2 * W, CoutT)
        r1 = y[:, :, 2:4, :].reshape(H, 2 * W, CoutT)
        up = jnp.stack([r0, r1], axis=1).reshape(2 * H, 2 * W, CoutT)

        sscale, sshift = _norm_params(sks, skss, skg, skb, cnt_s)
        _, Hs, Ws, Cs = skr.shape
        sk = jnp.maximum(skr[...].reshape(Hs, Ws, Cs).astype(_F32) * sscale
                         + sshift, 0.0)
        if bilinear:
            sk = jnp.dot(a_ref[...], sk.reshape(Hs * Ws, Cs),
                         preferred_element_type=_F32)
            sk = sk.reshape(2 * H, 2 * W, Cs)
        acc = _conv3x3([up, sk], [pad0, pad1], [w0, w1])
        _write_out(acc, o_ref, so_ref, sso_ref, 2 * H, 2 * W)
    return body


def _make_head(cnt):
    def body(xr, s_ref, ss_ref, g_ref, b_ref, w_ref, bias_ref,
             out_ref, feat_ref):
        _, H, W, C = xr.shape
        scale, shift = _norm_params(s_ref, ss_ref, g_ref, b_ref, cnt)
        feat = jnp.maximum(xr[...].reshape(H * W, C).astype(_F32) * scale + shift,
                           0.0)
        feat_ref[...] = feat.T.reshape(1, C, H * W)
        y = jax.lax.dot_general(w_ref[...], feat, (((1,), (1,)), ((), ())),
                                preferred_element_type=_F32) + bias_ref[...]
        out_ref[...] = y.reshape(1, y.shape[0], H * W)
    return body


def _prep_w3x3(w, cins):
    out, off = [], 0
    for ci in cins:
        wi = w[:, off:off + ci]
        out.append(jnp.transpose(wi, (2, 3, 1, 0))
                   .reshape(9 * ci, w.shape[0]).astype(_DOT))
        off += ci
    return out


def _stat_specs(N, C):
    return [pl.BlockSpec((N, 1, C), lambda n: (0, 0, 0)),
            pl.BlockSpec((N, 1, C), lambda n: (0, 0, 0)),
            pl.BlockSpec((1, C), lambda n: (0, 0)),
            pl.BlockSpec((1, C), lambda n: (0, 0))]


def _out_shapes(N, H, W, Cout, store):
    return (jax.ShapeDtypeStruct((N, H, W, Cout), store),
            jax.ShapeDtypeStruct((N, 1, Cout), _F32),
            jax.ShapeDtypeStruct((N, 1, Cout), _F32))


def _call_first_conv(x, w, store=_F32):
    N, H, W, C = x.shape
    Cout = int(w.shape[0])
    (w2,) = _prep_w3x3(w, [C])
    return pl.pallas_call(
        _k_first_conv,
        out_shape=_out_shapes(N, H, W, Cout, store),
        grid=(N,),
        in_specs=[pl.BlockSpec((1, H, W, C), lambda n: (n, 0, 0, 0)),
                  pl.BlockSpec((9 * C, Cout), lambda n: (0, 0))],
        out_specs=(pl.BlockSpec((1, H, W, Cout), lambda n: (n, 0, 0, 0)),
                   pl.BlockSpec((1, 1, Cout), lambda n: (n, 0, 0)),
                   pl.BlockSpec((1, 1, Cout), lambda n: (n, 0, 0))),
        scratch_shapes=[pltpu.VMEM((H + 2, W, 3 * C), _DOT)],
        compiler_params=_cparams(),
    )(x, w2)


def _call_norm_conv(xr, stats, g, be, w, pool_hw=None, store=_F32):
    s, ss = stats
    N, H, W, C = xr.shape
    Cout = int(w.shape[0])
    cnt = float(N * H * W)
    Ho, Wo = pool_hw if pool_hw is not None else (H, W)
    (w2,) = _prep_w3x3(w, [C])
    return pl.pallas_call(
        _make_norm_conv(pool_hw, cnt),
        out_shape=_out_shapes(N, Ho, Wo, Cout, store),
        grid=(N,),
        in_specs=[pl.BlockSpec((1, H, W, C), lambda n: (n, 0, 0, 0))]
                 + _stat_specs(N, C)
                 + [pl.BlockSpec((9 * C, Cout), lambda n: (0, 0))],
        out_specs=(pl.BlockSpec((1, Ho, Wo, Cout), lambda n: (n, 0, 0, 0)),
                   pl.BlockSpec((1, 1, Cout), lambda n: (n, 0, 0)),
                   pl.BlockSpec((1, 1, Cout), lambda n: (n, 0, 0))),
        scratch_shapes=[pltpu.VMEM((Ho + 2, Wo, 3 * C), _DOT)],
        compiler_params=_cparams(),
    )(xr, s, ss, g.reshape(1, C), be.reshape(1, C), w2)


def _interp_matrix(out_size, in_size):
    if out_size == in_size:
        return np.eye(in_size, dtype=np.float32)
    o = np.arange(out_size, dtype=np.float64)
    src = np.maximum((o + 0.5) * (in_size / out_size) - 0.5, 0.0)
    i0 = np.clip(np.floor(src).astype(np.int64), 0, in_size - 1)
    i1 = np.minimum(i0 + 1, in_size - 1)
    l1 = src - i0
    m = np.zeros((out_size, in_size), dtype=np.float64)
    rows = np.arange(out_size)
    m[rows, i0] += 1.0 - l1
    m[rows, i1] += l1
    return m.astype(np.float32)


def _call_dec_fused(xr, xstats, xg, xbe, up_w, up_b, skr, sstats, sg, sbe, w,
                    store=_F32):
    N, H, W, Cin = xr.shape
    CoutT = int(up_w.shape[1])
    Ho, Wo = 2 * H, 2 * W
    _, Hs, Ws, Cs = skr.shape
    Cout = int(w.shape[0])
    bilinear = (Hs, Ws) != (Ho, Wo)
    cnt_x = float(N * H * W)
    cnt_s = float(N * Hs * Ws)
    wT = jnp.transpose(up_w, (0, 2, 3, 1)).reshape(Cin, 4 * CoutT).astype(_DOT)
    bT = jnp.tile(up_b.reshape(1, CoutT), (1, 4))
    w0, w1 = _prep_w3x3(w, [CoutT, Cs])

    args = [xr, xstats[0], xstats[1], xg.reshape(1, Cin), xbe.reshape(1, Cin),
            wT, bT,
            skr, sstats[0], sstats[1], sg.reshape(1, Cs), sbe.reshape(1, Cs)]
    in_specs = ([pl.BlockSpec((1, H, W, Cin), lambda n: (n, 0, 0, 0))]
                + _stat_specs(N, Cin)
                + [pl.BlockSpec((Cin, 4 * CoutT), lambda n: (0, 0)),
                   pl.BlockSpec((1, 4 * CoutT), lambda n: (0, 0)),
                   pl.BlockSpec((1, Hs, Ws, Cs), lambda n: (n, 0, 0, 0))]
                + _stat_specs(N, Cs))
    if bilinear:
        a = jnp.asarray(np.kron(_interp_matrix(Ho, Hs), _interp_matrix(Wo, Ws)))
        args.append(a)
        in_specs.append(pl.BlockSpec((Ho * Wo, Hs * Ws), lambda n: (0, 0)))
    args += [w0, w1]
    in_specs += [pl.BlockSpec((9 * CoutT, Cout), lambda n: (0, 0)),
                 pl.BlockSpec((9 * Cs, Cout), lambda n: (0, 0))]

    return pl.pallas_call(
        _make_dec_fused(bilinear, cnt_x, cnt_s),
        out_shape=_out_shapes(N, Ho, Wo, Cout, store),
        grid=(N,),
        in_specs=in_specs,
        out_specs=(pl.BlockSpec((1, Ho, Wo, Cout), lambda n: (n, 0, 0, 0)),
                   pl.BlockSpec((1, 1, Cout), lambda n: (n, 0, 0)),
                   pl.BlockSpec((1, 1, Cout), lambda n: (n, 0, 0))),
        scratch_shapes=[pltpu.VMEM((Ho + 2, Wo, 3 * CoutT), _DOT),
                        pltpu.VMEM((Ho + 2, Wo, 3 * Cs), _DOT)],
        compiler_params=_cparams(),
    )(*args)


def _call_head(xr, stats, g, be, w, b):
    s, ss = stats
    N, H, W, C = xr.shape
    Cout = int(w.shape[0])
    cnt = float(N * H * W)
    return pl.pallas_call(
        _make_head(cnt),
        out_shape=(jax.ShapeDtypeStruct((N, Cout, H * W), _F32),
                   jax.ShapeDtypeStruct((N, C, H * W), _F32)),
        grid=(N,),
        in_specs=[pl.BlockSpec((1, H, W, C), lambda n: (n, 0, 0, 0))]
                 + _stat_specs(N, C)
                 + [pl.BlockSpec((Cout, C), lambda n: (0, 0)),
                    pl.BlockSpec((Cout, 1), lambda n: (0, 0))],
        out_specs=(pl.BlockSpec((1, Cout, H * W), lambda n: (n, 0, 0)),
                   pl.BlockSpec((1, C, H * W), lambda n: (n, 0, 0))),
        compiler_params=_cparams(),
    )(xr, s, ss, g.reshape(1, C), be.reshape(1, C),
      w[:, :, 0, 0], b.reshape(Cout, 1))


def _forward(x,
             enc0_w1, enc0_g1, enc0_be1, enc0_w2, enc0_g2, enc0_be2,
             enc1_w1, enc1_g1, enc1_be1, enc1_w2, enc1_g2, enc1_be2,
             bott_w1, bott_g1, bott_be1, bott_w2, bott_g2, bott_be2,
             dec0_up_w, dec0_up_b,
             dec0_conv_w1, dec0_conv_g1, dec0_conv_be1,
             dec0_conv_w2, dec0_conv_g2, dec0_conv_be2,
             dec1_up_w, dec1_up_b,
             dec1_conv_w1, dec1_conv_g1, dec1_conv_be1,
             dec1_conv_w2, dec1_conv_g2, dec1_conv_be2,
             final_w, final_b, target_hw):
    N = x.shape[0]
    xh = jnp.transpose(x, (0, 2, 3, 1)).astype(_F32)

    o1, s1, ss1 = _call_first_conv(xh, enc0_w1)
    o2, s2, ss2 = _call_norm_conv(o1, (s1, ss1), enc0_g1, enc0_be1, enc0_w2)
    o3, s3, ss3 = _call_norm_conv(o2, (s2, ss2), enc0_g2, enc0_be2, enc1_w1,
                                  pool_hw=target_hw)
    o4, s4, ss4 = _call_norm_conv(o3, (s3, ss3), enc1_g1, enc1_be1, enc1_w2)
    o5, s5, ss5 = _call_norm_conv(o4, (s4, ss4), enc1_g2, enc1_be2, bott_w1)
    o6, s6, ss6 = _call_norm_conv(o5, (s5, ss5), bott_g1, bott_be1, bott_w2)
    o7, s7, ss7 = _call_dec_fused(o6, (s6, ss6), bott_g2, bott_be2,
                                  dec0_up_w, dec0_up_b,
                                  o4, (s4, ss4), enc1_g2, enc1_be2, dec0_conv_w1)
    o8, s8, ss8 = _call_norm_conv(o7, (s7, ss7), dec0_conv_g1, dec0_conv_be1,
                                  dec0_conv_w2)
    o9, s9, ss9 = _call_dec_fused(o8, (s8, ss8), dec0_conv_g2, dec0_conv_be2,
                                  dec1_up_w, dec1_up_b,
                                  o2, (s2, ss2), enc0_g2, enc0_be2, dec1_conv_w1)
    o10, s10, ss10 = _call_norm_conv(o9, (s9, ss9), dec1_conv_g1, dec1_conv_be1,
                                     dec1_conv_w2)
    out, feat = _call_head(o10, (s10, ss10), dec1_conv_g2, dec1_conv_be2,
                           final_w, final_b)
    _, H, W, _ = o10.shape
    Cf = feat.shape[1]
    return (out.reshape(N, out.shape[1], H, W), feat.reshape(N, Cf, H, W))


def kernel(x, enc0_w1, enc0_g1, enc0_be1, enc0_w2, enc0_g2, enc0_be2,
           enc1_w1, enc1_g1, enc1_be1, enc1_w2, enc1_g2, enc1_be2,
           bott_w1, bott_g1, bott_be1, bott_w2, bott_g2, bott_be2,
           dec0_up_w, dec0_up_b,
           dec0_conv_w1, dec0_conv_g1, dec0_conv_be1,
           dec0_conv_w2, dec0_conv_g2, dec0_conv_be2,
           dec1_up_w, dec1_up_b,
           dec1_conv_w1, dec1_conv_g1, dec1_conv_be1,
           dec1_conv_w2, dec1_conv_g2, dec1_conv_be2,
           final_w, final_b):
    return _forward(x,
                    enc0_w1, enc0_g1, enc0_be1, enc0_w2, enc0_g2, enc0_be2,
                    enc1_w1, enc1_g1, enc1_be1, enc1_w2, enc1_g2, enc1_be2,
                    bott_w1, bott_g1, bott_be1, bott_w2, bott_g2, bott_be2,
                    dec0_up_w, dec0_up_b,
                    dec0_conv_w1, dec0_conv_g1, dec0_conv_be1,
                    dec0_conv_w2, dec0_conv_g2, dec0_conv_be2,
                    dec1_up_w, dec1_up_b,
                    dec1_conv_w1, dec1_conv_g1, dec1_conv_be1,
                    dec1_conv_w2, dec1_conv_g2, dec1_conv_be2,
                    final_w, final_b, target_hw=(16, 16))


# speedup vs baseline: 1.0536x; 1.0536x over previous
import numpy as np
import jax
import jax.numpy as jnp
from jax.experimental import pallas as pl
from jax.experimental.pallas import tpu as pltpu

_EPS = 1e-5
_DOT = jnp.bfloat16
_F32 = jnp.float32


def _cparams():
    return pltpu.CompilerParams(
        dimension_semantics=("parallel",),
        vmem_limit_bytes=48 * 1024 * 1024,
    )


def _norm_params(s_ref, ss_ref, g_ref, b_ref, cnt):
    mean = jnp.sum(s_ref[...], axis=(0, 1)) / cnt
    var = jnp.maximum(jnp.sum(ss_ref[...], axis=(0, 1)) / cnt - mean * mean, 0.0)
    scale = g_ref[...].reshape(-1) * jax.lax.rsqrt(var + _EPS)
    shift = b_ref[...].reshape(-1) - mean * scale
    return scale, shift


def _conv3x3(vals, pad_refs, w_refs):
    H, W, _ = vals[0].shape
    acc = None
    for v, pad_ref, w_ref in zip(vals, pad_refs, w_refs):
        C = v.shape[-1]
        pad_ref[0:1, :, :] = jnp.zeros((1, W + 2, C), pad_ref.dtype)
        pad_ref[H + 1:H + 2, :, :] = jnp.zeros((1, W + 2, C), pad_ref.dtype)
        pad_ref[:, 0:1, :] = jnp.zeros((H + 2, 1, C), pad_ref.dtype)
        pad_ref[:, W + 1:W + 2, :] = jnp.zeros((H + 2, 1, C), pad_ref.dtype)
        pad_ref[1:H + 1, 1:W + 1, :] = v
        taps = [pad_ref[dy:dy + H, dx:dx + W, :]
                for dy in range(3) for dx in range(3)]
        patch = jnp.concatenate(taps, axis=-1).reshape(H * W, 9 * C).astype(_DOT)
        d = jnp.dot(patch, w_ref[...], preferred_element_type=_F32)
        acc = d if acc is None else acc + d
    return acc


def _write_out(acc, o_ref, so_ref, sso_ref, H, W):
    Cout = o_ref.shape[-1]
    o_ref[...] = acc.reshape(1, H, W, Cout).astype(o_ref.dtype)
    so_ref[...] = jnp.sum(acc, axis=0).reshape(1, 1, Cout)
    sso_ref[...] = jnp.sum(acc * acc, axis=0).reshape(1, 1, Cout)


def _k_first_conv(x_ref, w_ref, o_ref, so_ref, sso_ref, pad_ref):
    _, H, W, C = x_ref.shape
    acc = _conv3x3([x_ref[...].reshape(H, W, C)], [pad_ref], [w_ref])
    _write_out(acc, o_ref, so_ref, sso_ref, H, W)


def _make_norm_conv(pool_raw, cnt):
    def body(x_ref, s_ref, ss_ref, g_ref, b_ref, w_ref, *out_refs):
        o_ref, so_ref, sso_ref = out_refs[:3]
        pad_ref = out_refs[-1]
        _, H, W, C = x_ref.shape
        scale, shift = _norm_params(s_ref, ss_ref, g_ref, b_ref, cnt)
        y = jnp.maximum(x_ref[...].reshape(H, W, C).astype(_F32) * scale + shift,
                        0.0)
        acc = _conv3x3([y], [pad_ref], [w_ref])
        _write_out(acc, o_ref, so_ref, sso_ref, H, W)
        if pool_raw is not None:
            Hp, Wp = pool_raw
            mx_ref, mn_ref = out_refs[3], out_refs[4]
            Cout = o_ref.shape[-1]
            a3 = acc.reshape(H, W, Cout)
            mx = jnp.max(a3.reshape(Hp, H // Hp, W, Cout), axis=1)
            mx = jnp.max(mx.reshape(Hp, Wp, W // Wp, Cout), axis=2)
            mn = jnp.min(a3.reshape(Hp, H // Hp, W, Cout), axis=1)
            mn = jnp.min(mn.reshape(Hp, Wp, W // Wp, Cout), axis=2)
            mx_ref[...] = mx.reshape(1, Hp, Wp, Cout)
            mn_ref[...] = mn.reshape(1, Hp, Wp, Cout)
    return body


def _make_pool_select_conv(cnt):
    def body(mx_ref, mn_ref, s_ref, ss_ref, g_ref, b_ref, w_ref,
             o_ref, so_ref, sso_ref, pad_ref):
        _, H, W, C = mx_ref.shape
        scale, shift = _norm_params(s_ref, ss_ref, g_ref, b_ref, cnt)
        sel = jnp.where(scale > 0.0,
                        mx_ref[...].reshape(H, W, C),
                        mn_ref[...].reshape(H, W, C))
        y = jnp.maximum(sel * scale + shift, 0.0)
        acc = _conv3x3([y], [pad_ref], [w_ref])
        _write_out(acc, o_ref, so_ref, sso_ref, H, W)
    return body


def _make_dec_fused(bilinear, cnt_x, cnt_s):
    def body(*refs):
        (xr, xs, xss, xg, xb, wT, bT,
         skr, sks, skss, skg, skb) = refs[:12]
        i = 12
        if bilinear:
            a_ref = refs[i]
            i += 1
        w0, w1, o_ref, so_ref, sso_ref, pad0, pad1 = refs[i:i + 7]
        _, H, W, Cin = xr.shape
        CoutT = wT.shape[1] // 4
        scale, shift = _norm_params(xs, xss, xg, xb, cnt_x)
        xv = jnp.maximum(xr[...].reshape(H * W, Cin).astype(_F32) * scale + shift,
                         0.0)
        y = jnp.dot(xv.astype(_DOT), wT[...], preferred_element_type=_F32) + bT[...]
        y = y.reshape(H, W, 4, CoutT)
        r0 = y[:, :, 0:2, :].reshape(H, 2 * W, CoutT)
        r1 = y[:, :, 2:4, :].reshape(H, 2 * W, CoutT)
        up = jnp.stack([r0, r1], axis=1).reshape(2 * H, 2 * W, CoutT)

        sscale, sshift = _norm_params(sks, skss, skg, skb, cnt_s)
        _, Hs, Ws, Cs = skr.shape
        sk = jnp.maximum(skr[...].reshape(Hs, Ws, Cs).astype(_F32) * sscale
                         + sshift, 0.0)
        if bilinear:
            sk = jnp.dot(a_ref[...], sk.reshape(Hs * Ws, Cs),
                         preferred_element_type=_F32)
            sk = sk.reshape(2 * H, 2 * W, Cs)
        acc = _conv3x3([up, sk], [pad0, pad1], [w0, w1])
        _write_out(acc, o_ref, so_ref, sso_ref, 2 * H, 2 * W)
    return body


def _make_head(cnt):
    def body(xr, s_ref, ss_ref, g_ref, b_ref, w_ref, bias_ref,
             out_ref, feat_ref):
        _, H, W, C = xr.shape
        scale, shift = _norm_params(s_ref, ss_ref, g_ref, b_ref, cnt)
        feat = jnp.maximum(xr[...].reshape(H * W, C).astype(_F32) * scale + shift,
                           0.0)
        feat_ref[...] = feat.T.reshape(1, C, H * W)
        y = jax.lax.dot_general(w_ref[...], feat, (((1,), (1,)), ((), ())),
                                preferred_element_type=_F32) + bias_ref[...]
        out_ref[...] = y.reshape(1, y.shape[0], H * W)
    return body


def _prep_w3x3(w, cins):
    out, off = [], 0
    for ci in cins:
        wi = w[:, off:off + ci]
        out.append(jnp.transpose(wi, (2, 3, 1, 0))
                   .reshape(9 * ci, w.shape[0]).astype(_DOT))
        off += ci
    return out


def _stat_specs(N, C):
    return [pl.BlockSpec((N, 1, C), lambda n: (0, 0, 0)),
            pl.BlockSpec((N, 1, C), lambda n: (0, 0, 0)),
            pl.BlockSpec((1, C), lambda n: (0, 0)),
            pl.BlockSpec((1, C), lambda n: (0, 0))]


def _out_shapes(N, H, W, Cout, store):
    return (jax.ShapeDtypeStruct((N, H, W, Cout), store),
            jax.ShapeDtypeStruct((N, 1, Cout), _F32),
            jax.ShapeDtypeStruct((N, 1, Cout), _F32))


def _call_first_conv(x, w, store=_F32):
    N, H, W, C = x.shape
    Cout = int(w.shape[0])
    (w2,) = _prep_w3x3(w, [C])
    return pl.pallas_call(
        _k_first_conv,
        out_shape=_out_shapes(N, H, W, Cout, store),
        grid=(N,),
        in_specs=[pl.BlockSpec((1, H, W, C), lambda n: (n, 0, 0, 0)),
                  pl.BlockSpec((9 * C, Cout), lambda n: (0, 0))],
        out_specs=(pl.BlockSpec((1, H, W, Cout), lambda n: (n, 0, 0, 0)),
                   pl.BlockSpec((1, 1, Cout), lambda n: (n, 0, 0)),
                   pl.BlockSpec((1, 1, Cout), lambda n: (n, 0, 0))),
        scratch_shapes=[pltpu.VMEM((H + 2, W + 2, C), _F32)],
        compiler_params=_cparams(),
    )(x, w2)


def _call_norm_conv(xr, stats, g, be, w, pool_raw=None, store=_F32):
    s, ss = stats
    N, H, W, C = xr.shape
    Cout = int(w.shape[0])
    cnt = float(N * H * W)
    (w2,) = _prep_w3x3(w, [C])
    out_shape = list(_out_shapes(N, H, W, Cout, store))
    out_specs = [pl.BlockSpec((1, H, W, Cout), lambda n: (n, 0, 0, 0)),
                 pl.BlockSpec((1, 1, Cout), lambda n: (n, 0, 0)),
                 pl.BlockSpec((1, 1, Cout), lambda n: (n, 0, 0))]
    if pool_raw is not None:
        Hp, Wp = pool_raw
        for _ in range(2):
            out_shape.append(jax.ShapeDtypeStruct((N, Hp, Wp, Cout), _F32))
            out_specs.append(pl.BlockSpec((1, Hp, Wp, Cout),
                                          lambda n: (n, 0, 0, 0)))
    return pl.pallas_call(
        _make_norm_conv(pool_raw, cnt),
        out_shape=tuple(out_shape),
        grid=(N,),
        in_specs=[pl.BlockSpec((1, H, W, C), lambda n: (n, 0, 0, 0))]
                 + _stat_specs(N, C)
                 + [pl.BlockSpec((9 * C, Cout), lambda n: (0, 0))],
        out_specs=tuple(out_specs),
        scratch_shapes=[pltpu.VMEM((H + 2, W + 2, C), _F32)],
        compiler_params=_cparams(),
    )(xr, s, ss, g.reshape(1, C), be.reshape(1, C), w2)


def _call_pool_select_conv(mx, mn, stats, g, be, w, cnt, store=_F32):
    s, ss = stats
    N, H, W, C = mx.shape
    Cout = int(w.shape[0])
    (w2,) = _prep_w3x3(w, [C])
    return pl.pallas_call(
        _make_pool_select_conv(cnt),
        out_shape=_out_shapes(N, H, W, Cout, store),
        grid=(N,),
        in_specs=[pl.BlockSpec((1, H, W, C), lambda n: (n, 0, 0, 0)),
                  pl.BlockSpec((1, H, W, C), lambda n: (n, 0, 0, 0))]
                 + _stat_specs(N, C)
                 + [pl.BlockSpec((9 * C, Cout), lambda n: (0, 0))],
        out_specs=(pl.BlockSpec((1, H, W, Cout), lambda n: (n, 0, 0, 0)),
                   pl.BlockSpec((1, 1, Cout), lambda n: (n, 0, 0)),
                   pl.BlockSpec((1, 1, Cout), lambda n: (n, 0, 0))),
        scratch_shapes=[pltpu.VMEM((H + 2, W + 2, C), _F32)],
        compiler_params=_cparams(),
    )(mx, mn, s, ss, g.reshape(1, C), be.reshape(1, C), w2)


def _interp_matrix(out_size, in_size):
    if out_size == in_size:
        return np.eye(in_size, dtype=np.float32)
    o = np.arange(out_size, dtype=np.float64)
    src = np.maximum((o + 0.5) * (in_size / out_size) - 0.5, 0.0)
    i0 = np.clip(np.floor(src).astype(np.int64), 0, in_size - 1)
    i1 = np.minimum(i0 + 1, in_size - 1)
    l1 = src - i0
    m = np.zeros((out_size, in_size), dtype=np.float64)
    rows = np.arange(out_size)
    m[rows, i0] += 1.0 - l1
    m[rows, i1] += l1
    return m.astype(np.float32)


def _call_dec_fused(xr, xstats, xg, xbe, up_w, up_b, skr, sstats, sg, sbe, w,
                    store=_F32):
    N, H, W, Cin = xr.shape
    CoutT = int(up_w.shape[1])
    Ho, Wo = 2 * H, 2 * W
    _, Hs, Ws, Cs = skr.shape
    Cout = int(w.shape[0])
    bilinear = (Hs, Ws) != (Ho, Wo)
    cnt_x = float(N * H * W)
    cnt_s = float(N * Hs * Ws)
    wT = jnp.transpose(up_w, (0, 2, 3, 1)).reshape(Cin, 4 * CoutT).astype(_DOT)
    bT = jnp.tile(up_b.reshape(1, CoutT), (1, 4))
    w0, w1 = _prep_w3x3(w, [CoutT, Cs])

    args = [xr, xstats[0], xstats[1], xg.reshape(1, Cin), xbe.reshape(1, Cin),
            wT, bT,
            skr, sstats[0], sstats[1], sg.reshape(1, Cs), sbe.reshape(1, Cs)]
    in_specs = ([pl.BlockSpec((1, H, W, Cin), lambda n: (n, 0, 0, 0))]
                + _stat_specs(N, Cin)
                + [pl.BlockSpec((Cin, 4 * CoutT), lambda n: (0, 0)),
                   pl.BlockSpec((1, 4 * CoutT), lambda n: (0, 0)),
                   pl.BlockSpec((1, Hs, Ws, Cs), lambda n: (n, 0, 0, 0))]
                + _stat_specs(N, Cs))
    if bilinear:
        a = jnp.asarray(np.kron(_interp_matrix(Ho, Hs), _interp_matrix(Wo, Ws)))
        args.append(a)
        in_specs.append(pl.BlockSpec((Ho * Wo, Hs * Ws), lambda n: (0, 0)))
    args += [w0, w1]
    in_specs += [pl.BlockSpec((9 * CoutT, Cout), lambda n: (0, 0)),
                 pl.BlockSpec((9 * Cs, Cout), lambda n: (0, 0))]

    return pl.pallas_call(
        _make_dec_fused(bilinear, cnt_x, cnt_s),
        out_shape=_out_shapes(N, Ho, Wo, Cout, store),
        grid=(N,),
        in_specs=in_specs,
        out_specs=(pl.BlockSpec((1, Ho, Wo, Cout), lambda n: (n, 0, 0, 0)),
                   pl.BlockSpec((1, 1, Cout), lambda n: (n, 0, 0)),
                   pl.BlockSpec((1, 1, Cout), lambda n: (n, 0, 0))),
        scratch_shapes=[pltpu.VMEM((Ho + 2, Wo + 2, CoutT), _F32),
                        pltpu.VMEM((Ho + 2, Wo + 2, Cs), _F32)],
        compiler_params=_cparams(),
    )(*args)


def _call_head(xr, stats, g, be, w, b):
    s, ss = stats
    N, H, W, C = xr.shape
    Cout = int(w.shape[0])
    cnt = float(N * H * W)
    return pl.pallas_call(
        _make_head(cnt),
        out_shape=(jax.ShapeDtypeStruct((N, Cout, H * W), _F32),
                   jax.ShapeDtypeStruct((N, C, H * W), _F32)),
        grid=(N,),
        in_specs=[pl.BlockSpec((1, H, W, C), lambda n: (n, 0, 0, 0))]
                 + _stat_specs(N, C)
                 + [pl.BlockSpec((Cout, C), lambda n: (0, 0)),
                    pl.BlockSpec((Cout, 1), lambda n: (0, 0))],
        out_specs=(pl.BlockSpec((1, Cout, H * W), lambda n: (n, 0, 0)),
                   pl.BlockSpec((1, C, H * W), lambda n: (n, 0, 0))),
        compiler_params=_cparams(),
    )(xr, s, ss, g.reshape(1, C), be.reshape(1, C),
      w[:, :, 0, 0], b.reshape(Cout, 1))


def _forward(x,
             enc0_w1, enc0_g1, enc0_be1, enc0_w2, enc0_g2, enc0_be2,
             enc1_w1, enc1_g1, enc1_be1, enc1_w2, enc1_g2, enc1_be2,
             bott_w1, bott_g1, bott_be1, bott_w2, bott_g2, bott_be2,
             dec0_up_w, dec0_up_b,
             dec0_conv_w1, dec0_conv_g1, dec0_conv_be1,
             dec0_conv_w2, dec0_conv_g2, dec0_conv_be2,
             dec1_up_w, dec1_up_b,
             dec1_conv_w1, dec1_conv_g1, dec1_conv_be1,
             dec1_conv_w2, dec1_conv_g2, dec1_conv_be2,
             final_w, final_b, target_hw):
    N = x.shape[0]
    xh = jnp.transpose(x, (0, 2, 3, 1)).astype(_F32)

    o1, s1, ss1 = _call_first_conv(xh, enc0_w1)
    o2, s2, ss2, o2mx, o2mn = _call_norm_conv(o1, (s1, ss1), enc0_g1, enc0_be1,
                                              enc0_w2, pool_raw=target_hw)
    o3, s3, ss3 = _call_pool_select_conv(o2mx, o2mn, (s2, ss2),
                                         enc0_g2, enc0_be2, enc1_w1,
                                         cnt=float(N * o2.shape[1] * o2.shape[2]))
    o4, s4, ss4 = _call_norm_conv(o3, (s3, ss3), enc1_g1, enc1_be1, enc1_w2)
    o5, s5, ss5 = _call_norm_conv(o4, (s4, ss4), enc1_g2, enc1_be2, bott_w1)
    o6, s6, ss6 = _call_norm_conv(o5, (s5, ss5), bott_g1, bott_be1, bott_w2)
    o7, s7, ss7 = _call_dec_fused(o6, (s6, ss6), bott_g2, bott_be2,
                                  dec0_up_w, dec0_up_b,
                                  o4, (s4, ss4), enc1_g2, enc1_be2, dec0_conv_w1)
    o8, s8, ss8 = _call_norm_conv(o7, (s7, ss7), dec0_conv_g1, dec0_conv_be1,
                                  dec0_conv_w2)
    o9, s9, ss9 = _call_dec_fused(o8, (s8, ss8), dec0_conv_g2, dec0_conv_be2,
                                  dec1_up_w, dec1_up_b,
                                  o2, (s2, ss2), enc0_g2, enc0_be2, dec1_conv_w1)
    o10, s10, ss10 = _call_norm_conv(o9, (s9, ss9), dec1_conv_g1, dec1_conv_be1,
                                     dec1_conv_w2)
    out, feat = _call_head(o10, (s10, ss10), dec1_conv_g2, dec1_conv_be2,
                           final_w, final_b)
    _, H, W, _ = o10.shape
    Cf = feat.shape[1]
    return (out.reshape(N, out.shape[1], H, W), feat.reshape(N, Cf, H, W))


def kernel(x, enc0_w1, enc0_g1, enc0_be1, enc0_w2, enc0_g2, enc0_be2,
           enc1_w1, enc1_g1, enc1_be1, enc1_w2, enc1_g2, enc1_be2,
           bott_w1, bott_g1, bott_be1, bott_w2, bott_g2, bott_be2,
           dec0_up_w, dec0_up_b,
           dec0_conv_w1, dec0_conv_g1, dec0_conv_be1,
           dec0_conv_w2, dec0_conv_g2, dec0_conv_be2,
           dec1_up_w, dec1_up_b,
           dec1_conv_w1, dec1_conv_g1, dec1_conv_be1,
           dec1_conv_w2, dec1_conv_g2, dec1_conv_be2,
           final_w, final_b):
    return _forward(x,
                    enc0_w1, enc0_g1, enc0_be1, enc0_w2, enc0_g2, enc0_be2,
                    enc1_w1, enc1_g1, enc1_be1, enc1_w2, enc1_g2, enc1_be2,
                    bott_w1, bott_g1, bott_be1, bott_w2, bott_g2, bott_be2,
                    dec0_up_w, dec0_up_b,
                    dec0_conv_w1, dec0_conv_g1, dec0_conv_be1,
                    dec0_conv_w2, dec0_conv_g2, dec0_conv_be2,
                    dec1_up_w, dec1_up_b,
                    dec1_conv_w1, dec1_conv_g1, dec1_conv_be1,
                    dec1_conv_w2, dec1_conv_g2, dec1_conv_be2,
                    final_w, final_b, target_hw=(16, 16))


# speedup vs baseline: 1.0765x; 1.0217x over previous
import numpy as np
import jax
import jax.numpy as jnp
from jax.experimental import pallas as pl
from jax.experimental.pallas import tpu as pltpu

_EPS = 1e-5
_DOT = jnp.bfloat16
_F32 = jnp.float32


def _cparams():
    return pltpu.CompilerParams(
        dimension_semantics=("parallel",),
        vmem_limit_bytes=48 * 1024 * 1024,
    )


def _norm_params(s_ref, ss_ref, g_ref, b_ref, cnt):
    mean = jnp.sum(s_ref[...], axis=(0, 1)) / cnt
    var = jnp.maximum(jnp.sum(ss_ref[...], axis=(0, 1)) / cnt - mean * mean, 0.0)
    scale = g_ref[...].reshape(-1) * jax.lax.rsqrt(var + _EPS)
    shift = b_ref[...].reshape(-1) - mean * scale
    return scale, shift


def _conv3x3(vals, pad_refs, w_refs):
    H, W, _ = vals[0].shape
    acc = None
    for v, pad_ref, w_ref in zip(vals, pad_refs, w_refs):
        C = v.shape[-1]
        pad_ref[0:1, :, :] = jnp.zeros((1, W + 2, C), pad_ref.dtype)
        pad_ref[H + 1:H + 2, :, :] = jnp.zeros((1, W + 2, C), pad_ref.dtype)
        pad_ref[:, 0:1, :] = jnp.zeros((H + 2, 1, C), pad_ref.dtype)
        pad_ref[:, W + 1:W + 2, :] = jnp.zeros((H + 2, 1, C), pad_ref.dtype)
        pad_ref[1:H + 1, 1:W + 1, :] = v
        taps = [pad_ref[dy:dy + H, dx:dx + W, :]
                for dy in range(3) for dx in range(3)]
        patch = jnp.concatenate(taps, axis=-1).reshape(H * W, 9 * C).astype(_DOT)
        d = jnp.dot(patch, w_ref[...], preferred_element_type=_F32)
        acc = d if acc is None else acc + d
    return acc


def _write_out(acc, o_ref, so_ref, sso_ref, H, W):
    Cout = o_ref.shape[-1]
    o_ref[...] = acc.reshape(1, H, W, Cout).astype(o_ref.dtype)
    so_ref[...] = jnp.sum(acc, axis=0).reshape(1, 1, Cout)
    sso_ref[...] = jnp.sum(acc * acc, axis=0).reshape(1, 1, Cout)


def _k_first_conv(x_ref, w_ref, o_ref, so_ref, sso_ref, pad_ref):
    _, H, W, C = x_ref.shape
    acc = _conv3x3([x_ref[...].reshape(H, W, C)], [pad_ref], [w_ref])
    _write_out(acc, o_ref, so_ref, sso_ref, H, W)


def _conv3x3_b(v, pad_ref, w_ref):
    B, H, W, C = v.shape
    pad_ref[:, 0:1, :, :] = jnp.zeros((B, 1, W + 2, C), pad_ref.dtype)
    pad_ref[:, H + 1:H + 2, :, :] = jnp.zeros((B, 1, W + 2, C), pad_ref.dtype)
    pad_ref[:, :, 0:1, :] = jnp.zeros((B, H + 2, 1, C), pad_ref.dtype)
    pad_ref[:, :, W + 1:W + 2, :] = jnp.zeros((B, H + 2, 1, C), pad_ref.dtype)
    pad_ref[:, 1:H + 1, 1:W + 1, :] = v
    taps = [pad_ref[:, dy:dy + H, dx:dx + W, :]
            for dy in range(3) for dx in range(3)]
    patch = jnp.concatenate(taps, axis=-1).reshape(B * H * W, 9 * C).astype(_DOT)
    return jnp.dot(patch, w_ref[...], preferred_element_type=_F32)


def _make_norm_conv(pool_raw, cnt):
    def body(x_ref, s_ref, ss_ref, g_ref, b_ref, w_ref, *out_refs):
        o_ref, so_ref, sso_ref = out_refs[:3]
        pad_ref = out_refs[-1]
        B, H, W, C = x_ref.shape
        Cout = o_ref.shape[-1]
        scale, shift = _norm_params(s_ref, ss_ref, g_ref, b_ref, cnt)
        y = jnp.maximum(x_ref[...].astype(_F32) * scale + shift, 0.0)
        acc = _conv3x3_b(y, pad_ref, w_ref)
        o_ref[...] = acc.reshape(B, H, W, Cout).astype(o_ref.dtype)
        a2 = acc.reshape(B, H * W, Cout)
        so_ref[...] = jnp.sum(a2, axis=1, keepdims=True)
        sso_ref[...] = jnp.sum(a2 * a2, axis=1, keepdims=True)
        if pool_raw is not None:
            Hp, Wp = pool_raw
            mx_ref, mn_ref = out_refs[3], out_refs[4]
            Cout = o_ref.shape[-1]
            a3 = acc.reshape(H, W, Cout)
            mx = jnp.max(a3.reshape(Hp, H // Hp, W, Cout), axis=1)
            mx = jnp.max(mx.reshape(Hp, Wp, W // Wp, Cout), axis=2)
            mn = jnp.min(a3.reshape(Hp, H // Hp, W, Cout), axis=1)
            mn = jnp.min(mn.reshape(Hp, Wp, W // Wp, Cout), axis=2)
            mx_ref[...] = mx.reshape(1, Hp, Wp, Cout)
            mn_ref[...] = mn.reshape(1, Hp, Wp, Cout)
    return body


def _make_pool_select_conv(cnt):
    def body(mx_ref, mn_ref, s_ref, ss_ref, g_ref, b_ref, w_ref,
             o_ref, so_ref, sso_ref, pad_ref):
        _, H, W, C = mx_ref.shape
        scale, shift = _norm_params(s_ref, ss_ref, g_ref, b_ref, cnt)
        sel = jnp.where(scale > 0.0,
                        mx_ref[...].reshape(H, W, C),
                        mn_ref[...].reshape(H, W, C))
        y = jnp.maximum(sel * scale + shift, 0.0)
        acc = _conv3x3([y], [pad_ref], [w_ref])
        _write_out(acc, o_ref, so_ref, sso_ref, H, W)
    return body


def _make_dec_fused(bilinear, cnt_x, cnt_s):
    def body(*refs):
        (xr, xs, xss, xg, xb, wT, bT,
         skr, sks, skss, skg, skb) = refs[:12]
        i = 12
        if bilinear:
            a_ref = refs[i]
            i += 1
        w0, w1, o_ref, so_ref, sso_ref, pad0, pad1 = refs[i:i + 7]
        _, H, W, Cin = xr.shape
        CoutT = wT.shape[1] // 4
        scale, shift = _norm_params(xs, xss, xg, xb, cnt_x)
        xv = jnp.maximum(xr[...].reshape(H * W, Cin).astype(_F32) * scale + shift,
                         0.0)
        y = jnp.dot(xv.astype(_DOT), wT[...], preferred_element_type=_F32) + bT[...]
        y = y.reshape(H, W, 4, CoutT)
        r0 = y[:, :, 0:2, :].reshape(H, 2 * W, CoutT)
        r1 = y[:, :, 2:4, :].reshape(H, 2 * W, CoutT)
        up = jnp.stack([r0, r1], axis=1).reshape(2 * H, 2 * W, CoutT)

        sscale, sshift = _norm_params(sks, skss, skg, skb, cnt_s)
        _, Hs, Ws, Cs = skr.shape
        sk = jnp.maximum(skr[...].reshape(Hs, Ws, Cs).astype(_F32) * sscale
                         + sshift, 0.0)
        if bilinear:
            sk = jnp.dot(a_ref[...], sk.reshape(Hs * Ws, Cs),
                         preferred_element_type=_F32)
            sk = sk.reshape(2 * H, 2 * W, Cs)
        acc = _conv3x3([up, sk], [pad0, pad1], [w0, w1])
        _write_out(acc, o_ref, so_ref, sso_ref, 2 * H, 2 * W)
    return body


def _make_head(cnt):
    def body(xr, s_ref, ss_ref, g_ref, b_ref, w_ref, bias_ref,
             out_ref, feat_ref):
        _, H, W, C = xr.shape
        scale, shift = _norm_params(s_ref, ss_ref, g_ref, b_ref, cnt)
        feat = jnp.maximum(xr[...].reshape(H * W, C).astype(_F32) * scale + shift,
                           0.0)
        feat_ref[...] = feat.T.reshape(1, C, H * W)
        y = jax.lax.dot_general(w_ref[...], feat, (((1,), (1,)), ((), ())),
                                preferred_element_type=_F32) + bias_ref[...]
        out_ref[...] = y.reshape(1, y.shape[0], H * W)
    return body


def _prep_w3x3(w, cins):
    out, off = [], 0
    for ci in cins:
        wi = w[:, off:off + ci]
        out.append(jnp.transpose(wi, (2, 3, 1, 0))
                   .reshape(9 * ci, w.shape[0]).astype(_DOT))
        off += ci
    return out


def _stat_specs(N, C):
    return [pl.BlockSpec((N, 1, C), lambda n: (0, 0, 0)),
            pl.BlockSpec((N, 1, C), lambda n: (0, 0, 0)),
            pl.BlockSpec((1, C), lambda n: (0, 0)),
            pl.BlockSpec((1, C), lambda n: (0, 0))]


def _out_shapes(N, H, W, Cout, store):
    return (jax.ShapeDtypeStruct((N, H, W, Cout), store),
            jax.ShapeDtypeStruct((N, 1, Cout), _F32),
            jax.ShapeDtypeStruct((N, 1, Cout), _F32))


def _call_first_conv(x, w, store=_F32):
    N, H, W, C = x.shape
    Cout = int(w.shape[0])
    (w2,) = _prep_w3x3(w, [C])
    return pl.pallas_call(
        _k_first_conv,
        out_shape=_out_shapes(N, H, W, Cout, store),
        grid=(N,),
        in_specs=[pl.BlockSpec((1, H, W, C), lambda n: (n, 0, 0, 0)),
                  pl.BlockSpec((9 * C, Cout), lambda n: (0, 0))],
        out_specs=(pl.BlockSpec((1, H, W, Cout), lambda n: (n, 0, 0, 0)),
                   pl.BlockSpec((1, 1, Cout), lambda n: (n, 0, 0)),
                   pl.BlockSpec((1, 1, Cout), lambda n: (n, 0, 0))),
        scratch_shapes=[pltpu.VMEM((H + 2, W + 2, C), _F32)],
        compiler_params=_cparams(),
    )(x, w2)


def _call_norm_conv(xr, stats, g, be, w, pool_raw=None, store=_F32, bb=1):
    s, ss = stats
    N, H, W, C = xr.shape
    Cout = int(w.shape[0])
    cnt = float(N * H * W)
    (w2,) = _prep_w3x3(w, [C])
    out_shape = list(_out_shapes(N, H, W, Cout, store))
    out_specs = [pl.BlockSpec((bb, H, W, Cout), lambda n: (n, 0, 0, 0)),
                 pl.BlockSpec((bb, 1, Cout), lambda n: (n, 0, 0)),
                 pl.BlockSpec((bb, 1, Cout), lambda n: (n, 0, 0))]
    if pool_raw is not None:
        assert bb == 1
        Hp, Wp = pool_raw
        for _ in range(2):
            out_shape.append(jax.ShapeDtypeStruct((N, Hp, Wp, Cout), _F32))
            out_specs.append(pl.BlockSpec((1, Hp, Wp, Cout),
                                          lambda n: (n, 0, 0, 0)))
    return pl.pallas_call(
        _make_norm_conv(pool_raw, cnt),
        out_shape=tuple(out_shape),
        grid=(N // bb,),
        in_specs=[pl.BlockSpec((bb, H, W, C), lambda n: (n, 0, 0, 0))]
                 + _stat_specs(N, C)
                 + [pl.BlockSpec((9 * C, Cout), lambda n: (0, 0))],
        out_specs=tuple(out_specs),
        scratch_shapes=[pltpu.VMEM((bb, H + 2, W + 2, C), _F32)],
        compiler_params=_cparams(),
    )(xr, s, ss, g.reshape(1, C), be.reshape(1, C), w2)


def _call_pool_select_conv(mx, mn, stats, g, be, w, cnt, store=_F32):
    s, ss = stats
    N, H, W, C = mx.shape
    Cout = int(w.shape[0])
    (w2,) = _prep_w3x3(w, [C])
    return pl.pallas_call(
        _make_pool_select_conv(cnt),
        out_shape=_out_shapes(N, H, W, Cout, store),
        grid=(N,),
        in_specs=[pl.BlockSpec((1, H, W, C), lambda n: (n, 0, 0, 0)),
                  pl.BlockSpec((1, H, W, C), lambda n: (n, 0, 0, 0))]
                 + _stat_specs(N, C)
                 + [pl.BlockSpec((9 * C, Cout), lambda n: (0, 0))],
        out_specs=(pl.BlockSpec((1, H, W, Cout), lambda n: (n, 0, 0, 0)),
                   pl.BlockSpec((1, 1, Cout), lambda n: (n, 0, 0)),
                   pl.BlockSpec((1, 1, Cout), lambda n: (n, 0, 0))),
        scratch_shapes=[pltpu.VMEM((H + 2, W + 2, C), _F32)],
        compiler_params=_cparams(),
    )(mx, mn, s, ss, g.reshape(1, C), be.reshape(1, C), w2)


def _interp_matrix(out_size, in_size):
    if out_size == in_size:
        return np.eye(in_size, dtype=np.float32)
    o = np.arange(out_size, dtype=np.float64)
    src = np.maximum((o + 0.5) * (in_size / out_size) - 0.5, 0.0)
    i0 = np.clip(np.floor(src).astype(np.int64), 0, in_size - 1)
    i1 = np.minimum(i0 + 1, in_size - 1)
    l1 = src - i0
    m = np.zeros((out_size, in_size), dtype=np.float64)
    rows = np.arange(out_size)
    m[rows, i0] += 1.0 - l1
    m[rows, i1] += l1
    return m.astype(np.float32)


def _call_dec_fused(xr, xstats, xg, xbe, up_w, up_b, skr, sstats, sg, sbe, w,
                    store=_F32):
    N, H, W, Cin = xr.shape
    CoutT = int(up_w.shape[1])
    Ho, Wo = 2 * H, 2 * W
    _, Hs, Ws, Cs = skr.shape
    Cout = int(w.shape[0])
    bilinear = (Hs, Ws) != (Ho, Wo)
    cnt_x = float(N * H * W)
    cnt_s = float(N * Hs * Ws)
    wT = jnp.transpose(up_w, (0, 2, 3, 1)).reshape(Cin, 4 * CoutT).astype(_DOT)
    bT = jnp.tile(up_b.reshape(1, CoutT), (1, 4))
    w0, w1 = _prep_w3x3(w, [CoutT, Cs])

    args = [xr, xstats[0], xstats[1], xg.reshape(1, Cin), xbe.reshape(1, Cin),
            wT, bT,
            skr, sstats[0], sstats[1], sg.reshape(1, Cs), sbe.reshape(1, Cs)]
    in_specs = ([pl.BlockSpec((1, H, W, Cin), lambda n: (n, 0, 0, 0))]
                + _stat_specs(N, Cin)
                + [pl.BlockSpec((Cin, 4 * CoutT), lambda n: (0, 0)),
                   pl.BlockSpec((1, 4 * CoutT), lambda n: (0, 0)),
                   pl.BlockSpec((1, Hs, Ws, Cs), lambda n: (n, 0, 0, 0))]
                + _stat_specs(N, Cs))
    if bilinear:
        a = jnp.asarray(np.kron(_interp_matrix(Ho, Hs), _interp_matrix(Wo, Ws)))
        args.append(a)
        in_specs.append(pl.BlockSpec((Ho * Wo, Hs * Ws), lambda n: (0, 0)))
    args += [w0, w1]
    in_specs += [pl.BlockSpec((9 * CoutT, Cout), lambda n: (0, 0)),
                 pl.BlockSpec((9 * Cs, Cout), lambda n: (0, 0))]

    return pl.pallas_call(
        _make_dec_fused(bilinear, cnt_x, cnt_s),
        out_shape=_out_shapes(N, Ho, Wo, Cout, store),
        grid=(N,),
        in_specs=in_specs,
        out_specs=(pl.BlockSpec((1, Ho, Wo, Cout), lambda n: (n, 0, 0, 0)),
                   pl.BlockSpec((1, 1, Cout), lambda n: (n, 0, 0)),
                   pl.BlockSpec((1, 1, Cout), lambda n: (n, 0, 0))),
        scratch_shapes=[pltpu.VMEM((Ho + 2, Wo + 2, CoutT), _F32),
                        pltpu.VMEM((Ho + 2, Wo + 2, Cs), _F32)],
        compiler_params=_cparams(),
    )(*args)


def _call_head(xr, stats, g, be, w, b):
    s, ss = stats
    N, H, W, C = xr.shape
    Cout = int(w.shape[0])
    cnt = float(N * H * W)
    return pl.pallas_call(
        _make_head(cnt),
        out_shape=(jax.ShapeDtypeStruct((N, Cout, H * W), _F32),
                   jax.ShapeDtypeStruct((N, C, H * W), _F32)),
        grid=(N,),
        in_specs=[pl.BlockSpec((1, H, W, C), lambda n: (n, 0, 0, 0))]
                 + _stat_specs(N, C)
                 + [pl.BlockSpec((Cout, C), lambda n: (0, 0)),
                    pl.BlockSpec((Cout, 1), lambda n: (0, 0))],
        out_specs=(pl.BlockSpec((1, Cout, H * W), lambda n: (n, 0, 0)),
                   pl.BlockSpec((1, C, H * W), lambda n: (n, 0, 0))),
        compiler_params=_cparams(),
    )(xr, s, ss, g.reshape(1, C), be.reshape(1, C),
      w[:, :, 0, 0], b.reshape(Cout, 1))


def _forward(x,
             enc0_w1, enc0_g1, enc0_be1, enc0_w2, enc0_g2, enc0_be2,
             enc1_w1, enc1_g1, enc1_be1, enc1_w2, enc1_g2, enc1_be2,
             bott_w1, bott_g1, bott_be1, bott_w2, bott_g2, bott_be2,
             dec0_up_w, dec0_up_b,
             dec0_conv_w1, dec0_conv_g1, dec0_conv_be1,
             dec0_conv_w2, dec0_conv_g2, dec0_conv_be2,
             dec1_up_w, dec1_up_b,
             dec1_conv_w1, dec1_conv_g1, dec1_conv_be1,
             dec1_conv_w2, dec1_conv_g2, dec1_conv_be2,
             final_w, final_b, target_hw):
    N = x.shape[0]
    xh = jnp.transpose(x, (0, 2, 3, 1)).astype(_F32)

    o1, s1, ss1 = _call_first_conv(xh, enc0_w1)
    o2, s2, ss2, o2mx, o2mn = _call_norm_conv(o1, (s1, ss1), enc0_g1, enc0_be1,
                                              enc0_w2, pool_raw=target_hw)
    o3, s3, ss3 = _call_pool_select_conv(o2mx, o2mn, (s2, ss2),
                                         enc0_g2, enc0_be2, enc1_w1,
                                         cnt=float(N * o2.shape[1] * o2.shape[2]))
    bb = 4 if N % 4 == 0 else 1
    o4, s4, ss4 = _call_norm_conv(o3, (s3, ss3), enc1_g1, enc1_be1, enc1_w2,
                                  bb=bb)
    o5, s5, ss5 = _call_norm_conv(o4, (s4, ss4), enc1_g2, enc1_be2, bott_w1,
                                  bb=bb)
    o6, s6, ss6 = _call_norm_conv(o5, (s5, ss5), bott_g1, bott_be1, bott_w2,
                                  bb=bb)
    o7, s7, ss7 = _call_dec_fused(o6, (s6, ss6), bott_g2, bott_be2,
                                  dec0_up_w, dec0_up_b,
                                  o4, (s4, ss4), enc1_g2, enc1_be2, dec0_conv_w1)
    o8, s8, ss8 = _call_norm_conv(o7, (s7, ss7), dec0_conv_g1, dec0_conv_be1,
                                  dec0_conv_w2)
    o9, s9, ss9 = _call_dec_fused(o8, (s8, ss8), dec0_conv_g2, dec0_conv_be2,
                                  dec1_up_w, dec1_up_b,
                                  o2, (s2, ss2), enc0_g2, enc0_be2, dec1_conv_w1)
    o10, s10, ss10 = _call_norm_conv(o9, (s9, ss9), dec1_conv_g1, dec1_conv_be1,
                                     dec1_conv_w2)
    out, feat = _call_head(o10, (s10, ss10), dec1_conv_g2, dec1_conv_be2,
                           final_w, final_b)
    _, H, W, _ = o10.shape
    Cf = feat.shape[1]
    return (out.reshape(N, out.shape[1], H, W), feat.reshape(N, Cf, H, W))


def kernel(x, enc0_w1, enc0_g1, enc0_be1, enc0_w2, enc0_g2, enc0_be2,
           enc1_w1, enc1_g1, enc1_be1, enc1_w2, enc1_g2, enc1_be2,
           bott_w1, bott_g1, bott_be1, bott_w2, bott_g2, bott_be2,
           dec0_up_w, dec0_up_b,
           dec0_conv_w1, dec0_conv_g1, dec0_conv_be1,
           dec0_conv_w2, dec0_conv_g2, dec0_conv_be2,
           dec1_up_w, dec1_up_b,
           dec1_conv_w1, dec1_conv_g1, dec1_conv_be1,
           dec1_conv_w2, dec1_conv_g2, dec1_conv_be2,
           final_w, final_b):
    return _forward(x,
                    enc0_w1, enc0_g1, enc0_be1, enc0_w2, enc0_g2, enc0_be2,
                    enc1_w1, enc1_g1, enc1_be1, enc1_w2, enc1_g2, enc1_be2,
                    bott_w1, bott_g1, bott_be1, bott_w2, bott_g2, bott_be2,
                    dec0_up_w, dec0_up_b,
                    dec0_conv_w1, dec0_conv_g1, dec0_conv_be1,
                    dec0_conv_w2, dec0_conv_g2, dec0_conv_be2,
                    dec1_up_w, dec1_up_b,
                    dec1_conv_w1, dec1_conv_g1, dec1_conv_be1,
                    dec1_conv_w2, dec1_conv_g2, dec1_conv_be2,
                    final_w, final_b, target_hw=(16, 16))


# speedup vs baseline: 1.0826x; 1.0057x over previous
import numpy as np
import jax
import jax.numpy as jnp
from jax.experimental import pallas as pl
from jax.experimental.pallas import tpu as pltpu

_EPS = 1e-5
_DOT = jnp.bfloat16
_F32 = jnp.float32


def _cparams():
    return pltpu.CompilerParams(
        dimension_semantics=("parallel",),
        vmem_limit_bytes=48 * 1024 * 1024,
    )


def _norm_params(s_ref, ss_ref, g_ref, b_ref, cnt):
    mean = jnp.sum(s_ref[...], axis=(0, 1)) / cnt
    var = jnp.maximum(jnp.sum(ss_ref[...], axis=(0, 1)) / cnt - mean * mean, 0.0)
    scale = g_ref[...].reshape(-1) * jax.lax.rsqrt(var + _EPS)
    shift = b_ref[...].reshape(-1) - mean * scale
    return scale, shift


def _conv3x3(vals, pad_refs, w_refs):
    H, W, _ = vals[0].shape
    acc = None
    for v, pad_ref, w_ref in zip(vals, pad_refs, w_refs):
        C = v.shape[-1]
        pad_ref[0:1, :, :] = jnp.zeros((1, W + 2, C), pad_ref.dtype)
        pad_ref[H + 1:H + 2, :, :] = jnp.zeros((1, W + 2, C), pad_ref.dtype)
        pad_ref[:, 0:1, :] = jnp.zeros((H + 2, 1, C), pad_ref.dtype)
        pad_ref[:, W + 1:W + 2, :] = jnp.zeros((H + 2, 1, C), pad_ref.dtype)
        pad_ref[1:H + 1, 1:W + 1, :] = v
        taps = [pad_ref[dy:dy + H, dx:dx + W, :]
                for dy in range(3) for dx in range(3)]
        patch = jnp.concatenate(taps, axis=-1).reshape(H * W, 9 * C).astype(_DOT)
        d = jnp.dot(patch, w_ref[...], preferred_element_type=_F32)
        acc = d if acc is None else acc + d
    return acc


def _write_out(acc, o_ref, so_ref, sso_ref, H, W):
    Cout = o_ref.shape[-1]
    o_ref[...] = acc.reshape(1, H, W, Cout).astype(o_ref.dtype)
    so_ref[...] = jnp.sum(acc, axis=0).reshape(1, 1, Cout)
    sso_ref[...] = jnp.sum(acc * acc, axis=0).reshape(1, 1, Cout)


def _k_first_conv(x_ref, w_ref, o_ref, so_ref, sso_ref, pad_ref):
    _, H, W, C = x_ref.shape
    acc = _conv3x3([x_ref[...].reshape(H, W, C)], [pad_ref], [w_ref])
    _write_out(acc, o_ref, so_ref, sso_ref, H, W)


def _conv3x3_b(v, pad_ref, w_ref):
    B, H, W, C = v.shape
    pad_ref[:, 0:1, :, :] = jnp.zeros((B, 1, W + 2, C), pad_ref.dtype)
    pad_ref[:, H + 1:H + 2, :, :] = jnp.zeros((B, 1, W + 2, C), pad_ref.dtype)
    pad_ref[:, :, 0:1, :] = jnp.zeros((B, H + 2, 1, C), pad_ref.dtype)
    pad_ref[:, :, W + 1:W + 2, :] = jnp.zeros((B, H + 2, 1, C), pad_ref.dtype)
    pad_ref[:, 1:H + 1, 1:W + 1, :] = v
    taps = [pad_ref[:, dy:dy + H, dx:dx + W, :]
            for dy in range(3) for dx in range(3)]
    patch = jnp.concatenate(taps, axis=-1).reshape(B * H * W, 9 * C).astype(_DOT)
    return jnp.dot(patch, w_ref[...], preferred_element_type=_F32)


def _make_norm_conv(pool_raw, cnt):
    def body(x_ref, s_ref, ss_ref, g_ref, b_ref, w_ref, *out_refs):
        o_ref, so_ref, sso_ref = out_refs[:3]
        pad_ref = out_refs[-1]
        B, H, W, C = x_ref.shape
        Cout = o_ref.shape[-1]
        scale, shift = _norm_params(s_ref, ss_ref, g_ref, b_ref, cnt)
        y = jnp.maximum(x_ref[...].astype(_F32) * scale + shift, 0.0)
        acc = _conv3x3_b(y, pad_ref, w_ref)
        o_ref[...] = acc.reshape(B, H, W, Cout).astype(o_ref.dtype)
        a2 = acc.reshape(B, H * W, Cout)
        so_ref[...] = jnp.sum(a2, axis=1, keepdims=True)
        sso_ref[...] = jnp.sum(a2 * a2, axis=1, keepdims=True)
        if pool_raw is not None:
            Hp, Wp = pool_raw
            mx_ref, mn_ref = out_refs[3], out_refs[4]
            Cout = o_ref.shape[-1]
            a3 = acc.reshape(H, W, Cout)
            mx = jnp.max(a3.reshape(Hp, H // Hp, W, Cout), axis=1)
            mx = jnp.max(mx.reshape(Hp, Wp, W // Wp, Cout), axis=2)
            mn = jnp.min(a3.reshape(Hp, H // Hp, W, Cout), axis=1)
            mn = jnp.min(mn.reshape(Hp, Wp, W // Wp, Cout), axis=2)
            mx_ref[...] = mx.reshape(1, Hp, Wp, Cout)
            mn_ref[...] = mn.reshape(1, Hp, Wp, Cout)
    return body


def _make_pool_select_conv(cnt):
    def body(mx_ref, mn_ref, s_ref, ss_ref, g_ref, b_ref, w_ref,
             o_ref, so_ref, sso_ref, pad_ref):
        B, H, W, C = mx_ref.shape
        Cout = o_ref.shape[-1]
        scale, shift = _norm_params(s_ref, ss_ref, g_ref, b_ref, cnt)
        sel = jnp.where(scale > 0.0, mx_ref[...], mn_ref[...])
        y = jnp.maximum(sel * scale + shift, 0.0)
        acc = _conv3x3_b(y, pad_ref, w_ref)
        o_ref[...] = acc.reshape(B, H, W, Cout).astype(o_ref.dtype)
        a2 = acc.reshape(B, H * W, Cout)
        so_ref[...] = jnp.sum(a2, axis=1, keepdims=True)
        sso_ref[...] = jnp.sum(a2 * a2, axis=1, keepdims=True)
    return body


def _make_dec_fused(bilinear, cnt_x, cnt_s):
    def body(*refs):
        (xr, xs, xss, xg, xb, wT, bT,
         skr, sks, skss, skg, skb) = refs[:12]
        i = 12
        if bilinear:
            a_ref = refs[i]
            i += 1
        w0, w1, o_ref, so_ref, sso_ref, pad0, pad1 = refs[i:i + 7]
        _, H, W, Cin = xr.shape
        CoutT = wT.shape[1] // 4
        scale, shift = _norm_params(xs, xss, xg, xb, cnt_x)
        xv = jnp.maximum(xr[...].reshape(H * W, Cin).astype(_F32) * scale + shift,
                         0.0)
        y = jnp.dot(xv.astype(_DOT), wT[...], preferred_element_type=_F32) + bT[...]
        y = y.reshape(H, W, 4, CoutT)
        r0 = y[:, :, 0:2, :].reshape(H, 2 * W, CoutT)
        r1 = y[:, :, 2:4, :].reshape(H, 2 * W, CoutT)
        up = jnp.stack([r0, r1], axis=1).reshape(2 * H, 2 * W, CoutT)

        sscale, sshift = _norm_params(sks, skss, skg, skb, cnt_s)
        _, Hs, Ws, Cs = skr.shape
        sk = jnp.maximum(skr[...].reshape(Hs, Ws, Cs).astype(_F32) * sscale
                         + sshift, 0.0)
        if bilinear:
            sk = jnp.dot(a_ref[...], sk.reshape(Hs * Ws, Cs),
                         preferred_element_type=_F32)
            sk = sk.reshape(2 * H, 2 * W, Cs)
        acc = _conv3x3([up, sk], [pad0, pad1], [w0, w1])
        _write_out(acc, o_ref, so_ref, sso_ref, 2 * H, 2 * W)
    return body


def _make_head(cnt):
    def body(xr, s_ref, ss_ref, g_ref, b_ref, w_ref, bias_ref,
             out_ref, feat_ref):
        _, H, W, C = xr.shape
        scale, shift = _norm_params(s_ref, ss_ref, g_ref, b_ref, cnt)
        feat = jnp.maximum(xr[...].reshape(H * W, C).astype(_F32) * scale + shift,
                           0.0)
        feat_ref[...] = feat.T.reshape(1, C, H * W)
        y = jax.lax.dot_general(w_ref[...], feat, (((1,), (1,)), ((), ())),
                                preferred_element_type=_F32) + bias_ref[...]
        out_ref[...] = y.reshape(1, y.shape[0], H * W)
    return body


def _prep_w3x3(w, cins):
    out, off = [], 0
    for ci in cins:
        wi = w[:, off:off + ci]
        out.append(jnp.transpose(wi, (2, 3, 1, 0))
                   .reshape(9 * ci, w.shape[0]).astype(_DOT))
        off += ci
    return out


def _stat_specs(N, C):
    return [pl.BlockSpec((N, 1, C), lambda n: (0, 0, 0)),
            pl.BlockSpec((N, 1, C), lambda n: (0, 0, 0)),
            pl.BlockSpec((1, C), lambda n: (0, 0)),
            pl.BlockSpec((1, C), lambda n: (0, 0))]


def _out_shapes(N, H, W, Cout, store):
    return (jax.ShapeDtypeStruct((N, H, W, Cout), store),
            jax.ShapeDtypeStruct((N, 1, Cout), _F32),
            jax.ShapeDtypeStruct((N, 1, Cout), _F32))


def _call_first_conv(x, w, store=_F32):
    N, H, W, C = x.shape
    Cout = int(w.shape[0])
    (w2,) = _prep_w3x3(w, [C])
    return pl.pallas_call(
        _k_first_conv,
        out_shape=_out_shapes(N, H, W, Cout, store),
        grid=(N,),
        in_specs=[pl.BlockSpec((1, H, W, C), lambda n: (n, 0, 0, 0)),
                  pl.BlockSpec((9 * C, Cout), lambda n: (0, 0))],
        out_specs=(pl.BlockSpec((1, H, W, Cout), lambda n: (n, 0, 0, 0)),
                   pl.BlockSpec((1, 1, Cout), lambda n: (n, 0, 0)),
                   pl.BlockSpec((1, 1, Cout), lambda n: (n, 0, 0))),
        scratch_shapes=[pltpu.VMEM((H + 2, W + 2, C), _F32)],
        compiler_params=_cparams(),
    )(x, w2)


def _call_norm_conv(xr, stats, g, be, w, pool_raw=None, store=_F32, bb=1):
    s, ss = stats
    N, H, W, C = xr.shape
    Cout = int(w.shape[0])
    cnt = float(N * H * W)
    (w2,) = _prep_w3x3(w, [C])
    out_shape = list(_out_shapes(N, H, W, Cout, store))
    out_specs = [pl.BlockSpec((bb, H, W, Cout), lambda n: (n, 0, 0, 0)),
                 pl.BlockSpec((bb, 1, Cout), lambda n: (n, 0, 0)),
                 pl.BlockSpec((bb, 1, Cout), lambda n: (n, 0, 0))]
    if pool_raw is not None:
        assert bb == 1
        Hp, Wp = pool_raw
        for _ in range(2):
            out_shape.append(jax.ShapeDtypeStruct((N, Hp, Wp, Cout), _F32))
            out_specs.append(pl.BlockSpec((1, Hp, Wp, Cout),
                                          lambda n: (n, 0, 0, 0)))
    return pl.pallas_call(
        _make_norm_conv(pool_raw, cnt),
        out_shape=tuple(out_shape),
        grid=(N // bb,),
        in_specs=[pl.BlockSpec((bb, H, W, C), lambda n: (n, 0, 0, 0))]
                 + _stat_specs(N, C)
                 + [pl.BlockSpec((9 * C, Cout), lambda n: (0, 0))],
        out_specs=tuple(out_specs),
        scratch_shapes=[pltpu.VMEM((bb, H + 2, W + 2, C), _F32)],
        compiler_params=_cparams(),
    )(xr, s, ss, g.reshape(1, C), be.reshape(1, C), w2)


def _call_pool_select_conv(mx, mn, stats, g, be, w, cnt, store=_F32, bb=1):
    s, ss = stats
    N, H, W, C = mx.shape
    Cout = int(w.shape[0])
    (w2,) = _prep_w3x3(w, [C])
    return pl.pallas_call(
        _make_pool_select_conv(cnt),
        out_shape=_out_shapes(N, H, W, Cout, store),
        grid=(N // bb,),
        in_specs=[pl.BlockSpec((bb, H, W, C), lambda n: (n, 0, 0, 0)),
                  pl.BlockSpec((bb, H, W, C), lambda n: (n, 0, 0, 0))]
                 + _stat_specs(N, C)
                 + [pl.BlockSpec((9 * C, Cout), lambda n: (0, 0))],
        out_specs=(pl.BlockSpec((bb, H, W, Cout), lambda n: (n, 0, 0, 0)),
                   pl.BlockSpec((bb, 1, Cout), lambda n: (n, 0, 0)),
                   pl.BlockSpec((bb, 1, Cout), lambda n: (n, 0, 0))),
        scratch_shapes=[pltpu.VMEM((bb, H + 2, W + 2, C), _F32)],
        compiler_params=_cparams(),
    )(mx, mn, s, ss, g.reshape(1, C), be.reshape(1, C), w2)


def _interp_matrix(out_size, in_size):
    if out_size == in_size:
        return np.eye(in_size, dtype=np.float32)
    o = np.arange(out_size, dtype=np.float64)
    src = np.maximum((o + 0.5) * (in_size / out_size) - 0.5, 0.0)
    i0 = np.clip(np.floor(src).astype(np.int64), 0, in_size - 1)
    i1 = np.minimum(i0 + 1, in_size - 1)
    l1 = src - i0
    m = np.zeros((out_size, in_size), dtype=np.float64)
    rows = np.arange(out_size)
    m[rows, i0] += 1.0 - l1
    m[rows, i1] += l1
    return m.astype(np.float32)


def _call_dec_fused(xr, xstats, xg, xbe, up_w, up_b, skr, sstats, sg, sbe, w,
                    store=_F32):
    N, H, W, Cin = xr.shape
    CoutT = int(up_w.shape[1])
    Ho, Wo = 2 * H, 2 * W
    _, Hs, Ws, Cs = skr.shape
    Cout = int(w.shape[0])
    bilinear = (Hs, Ws) != (Ho, Wo)
    cnt_x = float(N * H * W)
    cnt_s = float(N * Hs * Ws)
    wT = jnp.transpose(up_w, (0, 2, 3, 1)).reshape(Cin, 4 * CoutT).astype(_DOT)
    bT = jnp.tile(up_b.reshape(1, CoutT), (1, 4))
    w0, w1 = _prep_w3x3(w, [CoutT, Cs])

    args = [xr, xstats[0], xstats[1], xg.reshape(1, Cin), xbe.reshape(1, Cin),
            wT, bT,
            skr, sstats[0], sstats[1], sg.reshape(1, Cs), sbe.reshape(1, Cs)]
    in_specs = ([pl.BlockSpec((1, H, W, Cin), lambda n: (n, 0, 0, 0))]
                + _stat_specs(N, Cin)
                + [pl.BlockSpec((Cin, 4 * CoutT), lambda n: (0, 0)),
                   pl.BlockSpec((1, 4 * CoutT), lambda n: (0, 0)),
                   pl.BlockSpec((1, Hs, Ws, Cs), lambda n: (n, 0, 0, 0))]
                + _stat_specs(N, Cs))
    if bilinear:
        a = jnp.asarray(np.kron(_interp_matrix(Ho, Hs), _interp_matrix(Wo, Ws)))
        args.append(a)
        in_specs.append(pl.BlockSpec((Ho * Wo, Hs * Ws), lambda n: (0, 0)))
    args += [w0, w1]
    in_specs += [pl.BlockSpec((9 * CoutT, Cout), lambda n: (0, 0)),
                 pl.BlockSpec((9 * Cs, Cout), lambda n: (0, 0))]

    return pl.pallas_call(
        _make_dec_fused(bilinear, cnt_x, cnt_s),
        out_shape=_out_shapes(N, Ho, Wo, Cout, store),
        grid=(N,),
        in_specs=in_specs,
        out_specs=(pl.BlockSpec((1, Ho, Wo, Cout), lambda n: (n, 0, 0, 0)),
                   pl.BlockSpec((1, 1, Cout), lambda n: (n, 0, 0)),
                   pl.BlockSpec((1, 1, Cout), lambda n: (n, 0, 0))),
        scratch_shapes=[pltpu.VMEM((Ho + 2, Wo + 2, CoutT), _F32),
                        pltpu.VMEM((Ho + 2, Wo + 2, Cs), _F32)],
        compiler_params=_cparams(),
    )(*args)


def _call_head(xr, stats, g, be, w, b):
    s, ss = stats
    N, H, W, C = xr.shape
    Cout = int(w.shape[0])
    cnt = float(N * H * W)
    return pl.pallas_call(
        _make_head(cnt),
        out_shape=(jax.ShapeDtypeStruct((N, Cout, H * W), _F32),
                   jax.ShapeDtypeStruct((N, C, H * W), _F32)),
        grid=(N,),
        in_specs=[pl.BlockSpec((1, H, W, C), lambda n: (n, 0, 0, 0))]
                 + _stat_specs(N, C)
                 + [pl.BlockSpec((Cout, C), lambda n: (0, 0)),
                    pl.BlockSpec((Cout, 1), lambda n: (0, 0))],
        out_specs=(pl.BlockSpec((1, Cout, H * W), lambda n: (n, 0, 0)),
                   pl.BlockSpec((1, C, H * W), lambda n: (n, 0, 0))),
        compiler_params=_cparams(),
    )(xr, s, ss, g.reshape(1, C), be.reshape(1, C),
      w[:, :, 0, 0], b.reshape(Cout, 1))


def _forward(x,
             enc0_w1, enc0_g1, enc0_be1, enc0_w2, enc0_g2, enc0_be2,
             enc1_w1, enc1_g1, enc1_be1, enc1_w2, enc1_g2, enc1_be2,
             bott_w1, bott_g1, bott_be1, bott_w2, bott_g2, bott_be2,
             dec0_up_w, dec0_up_b,
             dec0_conv_w1, dec0_conv_g1, dec0_conv_be1,
             dec0_conv_w2, dec0_conv_g2, dec0_conv_be2,
             dec1_up_w, dec1_up_b,
             dec1_conv_w1, dec1_conv_g1, dec1_conv_be1,
             dec1_conv_w2, dec1_conv_g2, dec1_conv_be2,
             final_w, final_b, target_hw):
    N = x.shape[0]
    xh = jnp.transpose(x, (0, 2, 3, 1)).astype(_F32)

    o1, s1, ss1 = _call_first_conv(xh, enc0_w1)
    o2, s2, ss2, o2mx, o2mn = _call_norm_conv(o1, (s1, ss1), enc0_g1, enc0_be1,
                                              enc0_w2, pool_raw=target_hw)
    bb = 4 if N % 4 == 0 else 1
    o3, s3, ss3 = _call_pool_select_conv(o2mx, o2mn, (s2, ss2),
                                         enc0_g2, enc0_be2, enc1_w1,
                                         cnt=float(N * o2.shape[1] * o2.shape[2]),
                                         bb=bb)
    o4, s4, ss4 = _call_norm_conv(o3, (s3, ss3), enc1_g1, enc1_be1, enc1_w2,
                                  bb=bb)
    o5, s5, ss5 = _call_norm_conv(o4, (s4, ss4), enc1_g2, enc1_be2, bott_w1,
                                  bb=bb)
    o6, s6, ss6 = _call_norm_conv(o5, (s5, ss5), bott_g1, bott_be1, bott_w2,
                                  bb=bb)
    o7, s7, ss7 = _call_dec_fused(o6, (s6, ss6), bott_g2, bott_be2,
                                  dec0_up_w, dec0_up_b,
                                  o4, (s4, ss4), enc1_g2, enc1_be2, dec0_conv_w1)
    o8, s8, ss8 = _call_norm_conv(o7, (s7, ss7), dec0_conv_g1, dec0_conv_be1,
                                  dec0_conv_w2)
    o9, s9, ss9 = _call_dec_fused(o8, (s8, ss8), dec0_conv_g2, dec0_conv_be2,
                                  dec1_up_w, dec1_up_b,
                                  o2, (s2, ss2), enc0_g2, enc0_be2, dec1_conv_w1)
    o10, s10, ss10 = _call_norm_conv(o9, (s9, ss9), dec1_conv_g1, dec1_conv_be1,
                                     dec1_conv_w2)
    out, feat = _call_head(o10, (s10, ss10), dec1_conv_g2, dec1_conv_be2,
                           final_w, final_b)
    _, H, W, _ = o10.shape
    Cf = feat.shape[1]
    return (out.reshape(N, out.shape[1], H, W), feat.reshape(N, Cf, H, W))


def kernel(x, enc0_w1, enc0_g1, enc0_be1, enc0_w2, enc0_g2, enc0_be2,
           enc1_w1, enc1_g1, enc1_be1, enc1_w2, enc1_g2, enc1_be2,
           bott_w1, bott_g1, bott_be1, bott_w2, bott_g2, bott_be2,
           dec0_up_w, dec0_up_b,
           dec0_conv_w1, dec0_conv_g1, dec0_conv_be1,
           dec0_conv_w2, dec0_conv_g2, dec0_conv_be2,
           dec1_up_w, dec1_up_b,
           dec1_conv_w1, dec1_conv_g1, dec1_conv_be1,
           dec1_conv_w2, dec1_conv_g2, dec1_conv_be2,
           final_w, final_b):
    return _forward(x,
                    enc0_w1, enc0_g1, enc0_be1, enc0_w2, enc0_g2, enc0_be2,
                    enc1_w1, enc1_g1, enc1_be1, enc1_w2, enc1_g2, enc1_be2,
                    bott_w1, bott_g1, bott_be1, bott_w2, bott_g2, bott_be2,
                    dec0_up_w, dec0_up_b,
                    dec0_conv_w1, dec0_conv_g1, dec0_conv_be1,
                    dec0_conv_w2, dec0_conv_g2, dec0_conv_be2,
                    dec1_up_w, dec1_up_b,
                    dec1_conv_w1, dec1_conv_g1, dec1_conv_be1,
                    dec1_conv_w2, dec1_conv_g2, dec1_conv_be2,
                    final_w, final_b, target_hw=(16, 16))
